# speedup vs baseline: 4.0001x; 4.0001x over previous
import jax
import jax.numpy as jnp
from jax import lax
from jax.experimental import pallas as pl

N_EXP = 256
TOPK = 6
SCALE = 1.5
TBLK = 256


def _gate_block(x_ref, w_ref, b_ref, wout_ref, iout_ref):
    scores = lax.dot_general(
        w_ref[...], x_ref[...],
        (((1,), (1,)), ((), ())),
        preferred_element_type=jnp.float32,
    )
    s = jnp.sqrt(jax.nn.softplus(scores))
    b = s + b_ref[...].reshape(N_EXP, 1)
    eidx = lax.broadcasted_iota(jnp.int32, (N_EXP, TBLK), 0)

    ws = []
    idxs = []
    bcur = b
    neg = jnp.float32(-jnp.inf)
    for _ in range(TOPK):
        m = jnp.max(bcur, axis=0, keepdims=True)
        ismax = bcur == m
        idx = jnp.min(jnp.where(ismax, eidx, N_EXP), axis=0)
        sel = eidx == idx[None, :]
        w = jnp.max(jnp.where(sel, s, neg), axis=0)
        bcur = jnp.where(sel, neg, bcur)
        ws.append(w)
        idxs.append(idx)

    wsum = ws[0] + ws[1] + ws[2] + ws[3] + ws[4] + ws[5]
    inv = SCALE / wsum
    wout_ref[...] = jnp.stack([w * inv for w in ws], axis=1)
    iout_ref[...] = jnp.stack(idxs, axis=1)


@jax.jit
def kernel(x, W, bias):
    n_tokens = x.shape[0]
    grid = (n_tokens // TBLK,)
    wout, iout = pl.pallas_call(
        _gate_block,
        grid=grid,
        in_specs=[
            pl.BlockSpec((TBLK, x.shape[1]), lambda i: (i, 0)),
            pl.BlockSpec((N_EXP, x.shape[1]), lambda i: (0, 0)),
            pl.BlockSpec((N_EXP,), lambda i: (0,)),
        ],
        out_specs=[
            pl.BlockSpec((TBLK, TOPK), lambda i: (i, 0)),
            pl.BlockSpec((TBLK, TOPK), lambda i: (i, 0)),
        ],
        out_shape=[
            jax.ShapeDtypeStruct((n_tokens, TOPK), jnp.float32),
            jax.ShapeDtypeStruct((n_tokens, TOPK), jnp.int32),
        ],
    )(x, W, bias)
    return (wout, iout)
